# Initial kernel scaffold; baseline (speedup 1.0000x reference)
import jax
import jax.numpy as jnp
from jax.experimental import pallas as pl


def kernel(X, edge_index, W, bias):
    raise NotImplementedError("write your pallas kernel here")


# speedup vs baseline: 8.5597x; 8.5597x over previous
import jax
import jax.numpy as jnp
from jax import lax
from jax.experimental import pallas as pl
from jax.experimental.pallas import tpu as pltpu
from jax.experimental.pallas import tpu_sc as plsc

D = 128
NC, NS = 2, 16
NW = NC * NS
NPAD = 10240
ZSTR = NPAD // NS
KSTR = NPAD // NW
CH = 128

_mesh = plsc.VectorSubcoreMesh(
    core_axis_name="c", subcore_axis_name="s", num_cores=NC, num_subcores=NS
)
_params = pltpu.CompilerParams(needs_layout_passes=False)


def _worker():
    c = lax.axis_index("c")
    s = lax.axis_index("s")
    return c, s, s * NC + c


def _fill_1d(ref, n, val):
    v16 = jnp.full((16,), val, jnp.float32)

    @pl.loop(0, n // 16)
    def _(i):
        ref[pl.ds(i * 16, 16)] = v16


def _make_scatter_kernel(nchunk, swap, with_counts):
    g = 1 if swap else 0
    outs = [jax.ShapeDtypeStruct((NC * NPAD, D), jnp.float32)]
    if with_counts:
        outs += [
            jax.ShapeDtypeStruct((NC * NPAD,), jnp.float32),
            jax.ShapeDtypeStruct((NC * NPAD,), jnp.float32),
        ]
    scratch = [
        pltpu.VMEM_SHARED((NPAD, D), jnp.float32),
        pltpu.VMEM((2, 2, CH), jnp.int32),
        pltpu.VMEM((CH, D), jnp.float32),
        pltpu.VMEM((CH, D), jnp.float32),
        pltpu.SemaphoreType.DMA,
        pltpu.SemaphoreType.DMA,
        pltpu.SemaphoreType.DMA,
        pltpu.SemaphoreType.DMA,
    ]
    if with_counts:
        scratch += [
            pltpu.VMEM_SHARED((NPAD,), jnp.float32),
            pltpu.VMEM_SHARED((NPAD,), jnp.float32),
            pltpu.VMEM((CH,), jnp.float32),
            pltpu.VMEM((ZSTR,), jnp.float32),
        ]

    def body(src_hbm, pidx_hbm, *rest):
        if with_counts:
            (epart, gcnt_out, scnt_out, acc, pix, rows0, rows1,
             isem0, isem1, gsem0, gsem1, gcnt, scnt, ones, zcnt) = rest
        else:
            (epart, acc, pix, rows0, rows1,
             isem0, isem1, gsem0, gsem1) = rest
        c, s, w = _worker()
        z16 = jnp.zeros((16,), jnp.float32)

        @pl.loop(0, CH * (D // 16))
        def _(t):
            rows0[t // (D // 16), pl.ds((t % (D // 16)) * 16, 16)] = z16

        @pl.loop(0, ZSTR // CH)
        def _(q):
            pltpu.sync_copy(rows0, acc.at[pl.ds(s * ZSTR + q * CH, CH)])

        if with_counts:
            _fill_1d(ones, CH, 1.0)
            _fill_1d(zcnt, ZSTR, 0.0)
            pltpu.sync_copy(zcnt, gcnt.at[pl.ds(s * ZSTR, ZSTR)])
            pltpu.sync_copy(zcnt, scnt.at[pl.ds(s * ZSTR, ZSTR)])

        plsc.subcore_barrier()

        bufs = ((rows0, isem0, gsem0), (rows1, isem1, gsem1))

        def load_idx(j, b):
            return pltpu.async_copy(pidx_hbm.at[w, j], pix.at[b],
                                    bufs[b][1])

        def gather(b):
            return pltpu.async_copy(src_hbm.at[pix.at[b, g]], bufs[b][0],
                                    bufs[b][2])

        def scat(b):
            pltpu.sync_copy(bufs[b][0], acc.at[pix.at[b, 1 - g]], add=True)
            if with_counts:
                pltpu.sync_copy(ones, gcnt.at[pix.at[b, g]], add=True)
                pltpu.sync_copy(ones, scnt.at[pix.at[b, 1 - g]], add=True)

        @pl.loop(0, nchunk // 2)
        def _(jj):
            j0 = jj * 2
            di0 = load_idx(j0, 0)
            di1 = load_idx(j0 + 1, 1)
            di0.wait()
            dg0 = gather(0)
            di1.wait()
            dg1 = gather(1)
            dg0.wait()
            scat(0)
            dg1.wait()
            scat(1)

        plsc.subcore_barrier()

        pltpu.sync_copy(
            acc.at[pl.ds(s * ZSTR, ZSTR)],
            epart.at[pl.ds(c * NPAD + s * ZSTR, ZSTR)],
        )
        if with_counts:
            pltpu.sync_copy(gcnt.at[pl.ds(s * ZSTR, ZSTR)],
                            gcnt_out.at[pl.ds(c * NPAD + s * ZSTR, ZSTR)])
            pltpu.sync_copy(scnt.at[pl.ds(s * ZSTR, ZSTR)],
                            scnt_out.at[pl.ds(c * NPAD + s * ZSTR, ZSTR)])

    return pl.kernel(
        body,
        out_type=tuple(outs) if with_counts else outs[0],
        mesh=_mesh,
        scratch_types=scratch,
        compiler_params=_params,
    )


def _make_combine_kernel():
    cstr = 512
    nws = NPAD // cstr
    outs = (
        jax.ShapeDtypeStruct((NPAD, D), jnp.float32),
        jax.ShapeDtypeStruct((NPAD,), jnp.float32),
    )
    scratch = [
        pltpu.VMEM((CH, D), jnp.float32),
        pltpu.VMEM((CH, D), jnp.float32),
        pltpu.VMEM((cstr,), jnp.float32),
        pltpu.VMEM((cstr,), jnp.float32),
        pltpu.VMEM((cstr,), jnp.float32),
    ]

    def body(epart, dpart, bpart, efeat, dinv, p0, p1, cbuf, binv, dbuf):
        _, _, w = _worker()
        e0 = w * cstr

        @pl.when(w < nws)
        def _():
            pltpu.sync_copy(bpart.at[pl.ds(e0, cstr)], binv)
            pltpu.sync_copy(dpart.at[pl.ds(e0, cstr)], dbuf)

            @pl.loop(0, NC - 1)
            def _(k):
                pltpu.sync_copy(bpart.at[pl.ds((k + 1) * NPAD + e0, cstr)],
                                cbuf)

                @pl.loop(0, cstr // 16)
                def _(i):
                    sl = pl.ds(i * 16, 16)
                    binv[sl] = binv[sl] + cbuf[sl]

                pltpu.sync_copy(dpart.at[pl.ds((k + 1) * NPAD + e0, cstr)],
                                cbuf)

                @pl.loop(0, cstr // 16)
                def _(i):
                    sl = pl.ds(i * 16, 16)
                    dbuf[sl] = dbuf[sl] + cbuf[sl]

            @pl.loop(0, cstr // 16)
            def _(i):
                sl = pl.ds(i * 16, 16)
                b = binv[sl]
                binv[sl] = jnp.where(b > 0.0, 1.0 / b, 0.0)
                d = dbuf[sl]
                dbuf[sl] = jnp.where(d > 0.0, 1.0 / d, 0.0)

            pltpu.sync_copy(dbuf, dinv.at[pl.ds(e0, cstr)])
            zi = jnp.zeros((16,), jnp.int32)

            @pl.loop(0, cstr // CH)
            def _(q):
                r0 = e0 + q * CH
                pltpu.sync_copy(epart.at[pl.ds(r0, CH)], p0)
                pltpu.sync_copy(epart.at[pl.ds(NPAD + r0, CH)], p1)

                @pl.loop(0, CH)
                def _(i):
                    bs = plsc.load_gather(binv, [zi + (q * CH + i)])
                    for k in range(D // 16):
                        sl = pl.ds(k * 16, 16)
                        p0[i, sl] = (p0[i, sl] + p1[i, sl]) * bs

                pltpu.sync_copy(p0, efeat.at[pl.ds(r0, CH)])

    return pl.kernel(body, out_type=outs, mesh=_mesh, scratch_types=scratch,
                     compiler_params=_params)


def _mm_body(x_ref, w_ref, o_ref):
    o_ref[...] = jnp.dot(
        x_ref[...], w_ref[...], preferred_element_type=jnp.float32
    )


def _out_body(q_ref, dinv_ref, b_ref, o_ref):
    r = (q_ref[0] + q_ref[1]) * dinv_ref[...] + b_ref[...]
    m = jnp.max(r, axis=1, keepdims=True)
    e = jnp.exp(r - m)
    o_ref[...] = e / jnp.sum(e, axis=1, keepdims=True)


def kernel(X, edge_index, W, bias):
    n, d_in = X.shape
    d_out = W.shape[1]
    e = edge_index.shape[1]
    epad = ((e + NW * CH - 1) // (NW * CH)) * (NW * CH)
    nchunk = epad // (NW * CH)

    Xp = jnp.concatenate(
        [X, jnp.zeros((NPAD - n, d_in), jnp.float32)], axis=0
    )
    pad_ids = n + (jnp.arange(epad - e, dtype=jnp.int32) % (NPAD - n))
    ei = jnp.concatenate(
        [edge_index, jnp.stack([pad_ids, pad_ids])], axis=1
    )
    pidx = jnp.stack(
        [ei[0].reshape(NW, nchunk, CH), ei[1].reshape(NW, nchunk, CH)],
        axis=2,
    )

    rb = 512
    x = pl.pallas_call(
        _mm_body,
        grid=(NPAD // rb,),
        in_specs=[
            pl.BlockSpec((rb, d_in), lambda i: (i, 0)),
            pl.BlockSpec((d_in, d_out), lambda i: (0, 0)),
        ],
        out_specs=pl.BlockSpec((rb, d_out), lambda i: (i, 0)),
        out_shape=jax.ShapeDtypeStruct((NPAD, d_out), jnp.float32),
    )(Xp, W)

    epart, dpart, bpart = _make_scatter_kernel(nchunk, False, True)(x, pidx)

    efeat, dinv = _make_combine_kernel()(epart, dpart, bpart)

    opart = _make_scatter_kernel(nchunk, True, False)(efeat, pidx)

    ob = 400
    q = opart.reshape(NC, NPAD, d_out)
    dinv_col = dinv.reshape(NPAD, 1)
    bias2 = bias.reshape(1, d_out)
    out = pl.pallas_call(
        _out_body,
        grid=(n // ob,),
        in_specs=[
            pl.BlockSpec((NC, ob, d_out), lambda i: (0, i, 0)),
            pl.BlockSpec((ob, 1), lambda i: (i, 0)),
            pl.BlockSpec((1, d_out), lambda i: (0, 0)),
        ],
        out_specs=pl.BlockSpec((ob, d_out), lambda i: (i, 0)),
        out_shape=jax.ShapeDtypeStruct((n, d_out), jnp.float32),
    )(q, dinv_col, bias2)
    return out
